# baseline (speedup 1.0000x reference)
import jax
import jax.numpy as jnp
from jax.experimental import pallas as pl
from jax.experimental.pallas import tpu as pltpu

_NLAYERS = 16
_HID = 1024
_AUG = 1025


def _rnn_kernel(markers_ref, data_ref, Wt_ref, b_ref, hid_ref, Wout_ref,
                bout_ref, out_ref, hout_ref, g_ref, Wbuf_ref, sem):
    def copies_layer(l):
        s = l % 3
        return (
            pltpu.make_async_copy(Wt_ref.at[0:512, l, :],
                                  Wbuf_ref.at[s, 0:512, :], sem.at[s, 0]),
            pltpu.make_async_copy(Wt_ref.at[512:_AUG, l, :],
                                  Wbuf_ref.at[s, 512:_AUG, :], sem.at[s, 1]),
        )

    def start_layer(l):
        @pl.when(markers_ref[0, l] > 0)
        def _():
            for c in copies_layer(l):
                c.start()

    start_layer(0)
    start_layer(1)
    g_ref[0:1, 1:_AUG] = hid_ref[...]

    def make_body(l):
        def body(_, i):
            g_ref[0:1, 0:1] = jnp.full((1, 1), data_ref[0, i], jnp.float32)
            t = jax.lax.dot_general(
                g_ref[...], Wbuf_ref[l % 3],
                (((1,), (0,)), ((), ())),
                preferred_element_type=jnp.float32)
            h_new = jnp.maximum(t + b_ref[l], 0.0)
            g_ref[0:1, 1:_AUG] = h_new
            return i + 1

        return body

    i = jnp.asarray(0, jnp.int32)
    for l in range(_NLAYERS):
        if l + 2 < _NLAYERS:
            start_layer(l + 2)
        cnt = markers_ref[0, l]

        @pl.when(cnt > 0)
        def _(l=l):
            for c in copies_layer(l):
                c.wait()
        body = make_body(l)

        def body2(k, j, body=body):
            return body(k, body(k, j))

        i = jax.lax.fori_loop(0, cnt // 2, body2, i, unroll=False)
        i = jax.lax.fori_loop(0, cnt % 2, body, i, unroll=False)

    h_fin = g_ref[0:1, 1:_AUG]
    logits = jax.lax.dot_general(
        h_fin, Wout_ref[...],
        (((1,), (1,)), ((), ())),
        preferred_element_type=jnp.float32) + bout_ref[...]
    m = jnp.max(logits)
    z = logits - m
    out_ref[...] = z - jnp.log(jnp.sum(jnp.exp(z)))
    hout_ref[...] = h_fin


@jax.jit
def kernel(input_data, markers, hidden, W, b, W_out, b_out):
    nlayers, hid, inpp1 = W.shape
    out_sz = W_out.shape[0]

    Wt = jnp.transpose(W, (2, 0, 1))
    b_row = b[:, None, :]
    bout_row = b_out[None, :]

    out_row, h_row = pl.pallas_call(
        _rnn_kernel,
        in_specs=[
            pl.BlockSpec(memory_space=pltpu.SMEM),
            pl.BlockSpec(memory_space=pltpu.SMEM),
            pl.BlockSpec(memory_space=pl.ANY),
            pl.BlockSpec(memory_space=pltpu.VMEM),
            pl.BlockSpec(memory_space=pltpu.VMEM),
            pl.BlockSpec(memory_space=pltpu.VMEM),
            pl.BlockSpec(memory_space=pltpu.VMEM),
        ],
        out_specs=[
            pl.BlockSpec(memory_space=pltpu.VMEM),
            pl.BlockSpec(memory_space=pltpu.VMEM),
        ],
        out_shape=[
            jax.ShapeDtypeStruct((1, out_sz), jnp.float32),
            jax.ShapeDtypeStruct((1, hid), jnp.float32),
        ],
        scratch_shapes=[
            pltpu.VMEM((1, inpp1), jnp.float32),
            pltpu.VMEM((3, inpp1, hid), jnp.float32),
            pltpu.SemaphoreType.DMA((3, 2)),
        ],
    )(markers, input_data, Wt, b_row, hidden, W_out, bout_row)

    return out_row, h_row


# speedup vs baseline: 1.0147x; 1.0147x over previous
import jax
import jax.numpy as jnp
from jax.experimental import pallas as pl
from jax.experimental.pallas import tpu as pltpu

_NLAYERS = 16
_HID = 1024
_AUG = 1025


def _rnn_kernel(markers_ref, data_ref, Wt_ref, b_ref, hid_ref, Wout_ref,
                bout_ref, out_ref, hout_ref, g_ref, Wbuf_ref, sem):
    def copy_layer(l):
        return pltpu.make_async_copy(
            Wt_ref.at[:, l, :], Wbuf_ref.at[l % 3], sem.at[l % 3])

    def start_layer(l):
        @pl.when(markers_ref[0, l] > 0)
        def _():
            copy_layer(l).start()

    start_layer(0)
    start_layer(1)
    g_ref[0:1, 1:_AUG] = hid_ref[...]

    def make_body(l):
        def body(_, i):
            g_ref[0:1, 0:1] = jnp.full((1, 1), data_ref[0, i], jnp.float32)
            t = jax.lax.dot_general(
                g_ref[...], Wbuf_ref[l % 3],
                (((1,), (0,)), ((), ())),
                preferred_element_type=jnp.float32)
            h_new = jnp.maximum(t + b_ref[l], 0.0)
            g_ref[0:1, 1:_AUG] = h_new
            return i + 1

        return body

    i = jnp.asarray(0, jnp.int32)
    for l in range(_NLAYERS):
        if l + 2 < _NLAYERS:
            start_layer(l + 2)
        cnt = markers_ref[0, l]

        @pl.when(cnt > 0)
        def _(l=l):
            copy_layer(l).wait()
        body = make_body(l)

        def body3(k, j, body=body):
            return body(k, body(k, body(k, j)))

        i = jax.lax.fori_loop(0, cnt // 3, body3, i, unroll=False)
        i = jax.lax.fori_loop(0, cnt % 3, body, i, unroll=False)

    h_fin = g_ref[0:1, 1:_AUG]
    logits = jax.lax.dot_general(
        h_fin, Wout_ref[...],
        (((1,), (1,)), ((), ())),
        preferred_element_type=jnp.float32) + bout_ref[...]
    m = jnp.max(logits)
    z = logits - m
    out_ref[...] = z - jnp.log(jnp.sum(jnp.exp(z)))
    hout_ref[...] = h_fin


@jax.jit
def kernel(input_data, markers, hidden, W, b, W_out, b_out):
    nlayers, hid, inpp1 = W.shape
    out_sz = W_out.shape[0]

    Wt = jnp.transpose(W, (2, 0, 1))
    b_row = b[:, None, :]
    bout_row = b_out[None, :]

    out_row, h_row = pl.pallas_call(
        _rnn_kernel,
        in_specs=[
            pl.BlockSpec(memory_space=pltpu.SMEM),
            pl.BlockSpec(memory_space=pltpu.SMEM),
            pl.BlockSpec(memory_space=pl.ANY),
            pl.BlockSpec(memory_space=pltpu.VMEM),
            pl.BlockSpec(memory_space=pltpu.VMEM),
            pl.BlockSpec(memory_space=pltpu.VMEM),
            pl.BlockSpec(memory_space=pltpu.VMEM),
        ],
        out_specs=[
            pl.BlockSpec(memory_space=pltpu.VMEM),
            pl.BlockSpec(memory_space=pltpu.VMEM),
        ],
        out_shape=[
            jax.ShapeDtypeStruct((1, out_sz), jnp.float32),
            jax.ShapeDtypeStruct((1, hid), jnp.float32),
        ],
        scratch_shapes=[
            pltpu.VMEM((1, inpp1), jnp.float32),
            pltpu.VMEM((3, inpp1, hid), jnp.float32),
            pltpu.SemaphoreType.DMA((3,)),
        ],
    )(markers, input_data, Wt, b_row, hidden, W_out, bout_row)

    return out_row, h_row
